# baseline (speedup 1.0000x reference)
import functools

import jax
import jax.numpy as jnp
from jax import lax
from jax.experimental import pallas as pl
from jax.experimental.pallas import tpu as pltpu
from jax.experimental.pallas import tpu_sc as plsc

N_ROWS = 16384
N_COLS = 2048
LANES = 16
NC, NS = 2, 16
NW = NC * NS
ROWS_PER_W = N_ROWS // NW
R = 2
NBUF = 8
NCHUNK = ROWS_PER_W // R

_mesh = plsc.VectorSubcoreMesh(
    core_axis_name="c", subcore_axis_name="s", num_cores=NC, num_subcores=NS
)


@functools.partial(
    pl.kernel,
    out_type=[
        jax.ShapeDtypeStruct((N_ROWS, N_COLS), jnp.float32),
        jax.ShapeDtypeStruct((N_ROWS,), jnp.float32),
    ],
    mesh=_mesh,
    scratch_types=[
        pltpu.VMEM((NBUF, R, N_COLS), jnp.float32),
        pltpu.VMEM((NBUF, R, N_COLS), jnp.float32),
        pltpu.VMEM((ROWS_PER_W,), jnp.float32),
        pltpu.VMEM_SHARED((NS, NBUF, R, N_COLS), jnp.float32),
        [pltpu.SemaphoreType.DMA] * NBUF,
        [pltpu.SemaphoreType.DMA] * NBUF,
        pltpu.SemaphoreType.DMA,
    ],
)
def _permute_sc(in_hbm, out_hbm, ld_hbm, inbuf, outbuf, zbuf, spm,
                s_in, s_out, s_ld):
    sid = lax.axis_index("s")
    wid = sid * NC + lax.axis_index("c")
    base = wid * ROWS_PER_W

    def in_slice(c):
        return in_hbm.at[pl.ds(base + c * R, R)]

    def out_slice(c):
        return out_hbm.at[pl.ds(base + c * R, R)]

    zero = jnp.zeros((LANES,), jnp.float32)
    for i in range(ROWS_PER_W // LANES):
        zbuf[pl.ds(i * LANES, LANES)] = zero
    pltpu.async_copy(zbuf, ld_hbm.at[pl.ds(base, ROWS_PER_W)], s_ld)

    for b in range(NBUF - 1):
        pltpu.async_copy(in_slice(b), inbuf.at[b], s_in[b])

    @pl.loop(0, NCHUNK, step=NBUF)
    def _(g):
        for b in range(NBUF):
            c = g + b

            @pl.when(c + NBUF - 1 < NCHUNK)
            def _():
                pltpu.async_copy(
                    in_slice(c + NBUF - 1),
                    inbuf.at[(b + NBUF - 1) % NBUF],
                    s_in[(b + NBUF - 1) % NBUF],
                )

            pltpu.make_async_copy(in_slice(c), inbuf.at[b], s_in[b]).wait()

            @pl.when(c >= NBUF)
            def _():
                pltpu.make_async_copy(spm.at[sid, b], out_slice(c), s_out[b]).wait()

            @pl.loop(0, R)
            def _(r):
                @plsc.parallel_loop(0, N_COLS // LANES, unroll=8)
                def _(j):
                    x = inbuf[b, r, pl.ds(N_COLS - LANES - LANES * j, LANES)]
                    outbuf[b, r, pl.ds(LANES * j, LANES)] = lax.rev(x, (0,))

            pltpu.sync_copy(outbuf.at[b], spm.at[sid, b])
            pltpu.async_copy(spm.at[sid, b], out_slice(c), s_out[b])

    for b in range(NBUF):
        pltpu.make_async_copy(spm.at[sid, b], out_slice(0), s_out[b]).wait()
    pltpu.make_async_copy(zbuf, ld_hbm.at[pl.ds(base, ROWS_PER_W)], s_ld).wait()


def kernel(inputs, forward):
    out, logdet = _permute_sc(inputs)
    return (out, logdet.reshape(inputs.shape[0], 1))


# speedup vs baseline: 1.0539x; 1.0539x over previous
import functools

import jax
import jax.numpy as jnp
from jax import lax
from jax.experimental import pallas as pl
from jax.experimental.pallas import tpu as pltpu
from jax.experimental.pallas import tpu_sc as plsc

N_ROWS = 16384
N_COLS = 2048
LANES = 16
NC, NS = 2, 16
NW = NC * NS
ROWS_PER_W = N_ROWS // NW
R = 2
NBUF = 8
NCHUNK = ROWS_PER_W // R

_mesh = plsc.VectorSubcoreMesh(
    core_axis_name="c", subcore_axis_name="s", num_cores=NC, num_subcores=NS
)


@functools.partial(
    pl.kernel,
    out_type=[
        jax.ShapeDtypeStruct((N_ROWS, N_COLS), jnp.float32),
        jax.ShapeDtypeStruct((N_ROWS,), jnp.float32),
    ],
    mesh=_mesh,
    scratch_types=[
        pltpu.VMEM((NBUF, R, N_COLS), jnp.float32),
        pltpu.VMEM((NBUF, R, N_COLS), jnp.float32),
        pltpu.VMEM((ROWS_PER_W,), jnp.float32),
        pltpu.VMEM_SHARED((NS, NBUF, R, N_COLS), jnp.float32),
        [pltpu.SemaphoreType.DMA] * NBUF,
        [pltpu.SemaphoreType.DMA] * NBUF,
        pltpu.SemaphoreType.DMA,
    ],
)
def _permute_sc(in_hbm, out_hbm, ld_hbm, inbuf, outbuf, zbuf, spm,
                s_in, s_out, s_ld):
    sid = lax.axis_index("s")
    wid = sid * NC + lax.axis_index("c")
    base = wid * ROWS_PER_W

    def in_slice(c):
        return in_hbm.at[pl.ds(base + c * R, R)]

    def out_slice(c):
        return out_hbm.at[pl.ds(base + c * R, R)]

    zero = jnp.zeros((LANES,), jnp.float32)
    for i in range(ROWS_PER_W // LANES):
        zbuf[pl.ds(i * LANES, LANES)] = zero
    pltpu.async_copy(zbuf, ld_hbm.at[pl.ds(base, ROWS_PER_W)], s_ld)

    for b in range(NBUF - 1):
        pltpu.async_copy(in_slice(b), inbuf.at[b], s_in[b])

    @pl.loop(0, NCHUNK, step=NBUF)
    def _(g):
        for b in range(NBUF):
            c = g + b

            @pl.when(c + NBUF - 1 < NCHUNK)
            def _():
                pltpu.async_copy(
                    in_slice(c + NBUF - 1),
                    inbuf.at[(b + NBUF - 1) % NBUF],
                    s_in[(b + NBUF - 1) % NBUF],
                )

            pltpu.make_async_copy(in_slice(c), inbuf.at[b], s_in[b]).wait()

            @pl.when(c >= NBUF)
            def _():
                if b % 2 == 0:
                    pltpu.make_async_copy(outbuf.at[b], out_slice(c), s_out[b]).wait()
                else:
                    pltpu.make_async_copy(spm.at[sid, b], out_slice(c), s_out[b]).wait()

            @pl.loop(0, R)
            def _(r):
                @plsc.parallel_loop(0, N_COLS // LANES, unroll=8)
                def _(j):
                    x = inbuf[b, r, pl.ds(N_COLS - LANES - LANES * j, LANES)]
                    outbuf[b, r, pl.ds(LANES * j, LANES)] = lax.rev(x, (0,))

            if b % 2 == 0:
                pltpu.async_copy(outbuf.at[b], out_slice(c), s_out[b])
            else:
                pltpu.sync_copy(outbuf.at[b], spm.at[sid, b])
                pltpu.async_copy(spm.at[sid, b], out_slice(c), s_out[b])

    for b in range(NBUF):
        if b % 2 == 0:
            pltpu.make_async_copy(outbuf.at[b], out_slice(0), s_out[b]).wait()
        else:
            pltpu.make_async_copy(spm.at[sid, b], out_slice(0), s_out[b]).wait()
    pltpu.make_async_copy(zbuf, ld_hbm.at[pl.ds(base, ROWS_PER_W)], s_ld).wait()


def kernel(inputs, forward):
    out, logdet = _permute_sc(inputs)
    return (out, logdet.reshape(inputs.shape[0], 1))
